# baseline (speedup 1.0000x reference)
import functools

import jax
import jax.numpy as jnp
from jax import lax
from jax.experimental import pallas as pl
from jax.experimental.pallas import tpu as pltpu
from jax.experimental.pallas import tpu_sc as plsc

POOL = 1024
LEN = 16
K = 8
B, T, H = 4, 8192, 1024

TBLK = 128
NT = T // TBLK
OUT_T = K * LEN + T

NC = 2
NS = 16
NW = NC * NS
ROWS_PER_W = POOL // NW
LANES = 16


def _copy_mean_body(x_ref, out_ref, sum_ref):
    t = pl.program_id(0)
    xb = x_ref[...]
    out_ref[...] = xb

    @pl.when(t == 0)
    def _():
        sum_ref[...] = jnp.zeros_like(sum_ref)

    sum_ref[...] += jnp.sum(xb, axis=1)


def _pmean_sc_body(prompt_hbm, out_hbm, row_v, acc_v, sem):
    c = lax.axis_index("c")
    s = lax.axis_index("s")
    wid = s * NC + c
    base = wid * ROWS_PER_W

    def body(r, carry):
        pltpu.async_copy(prompt_hbm.at[base + r], row_v, sem).wait()
        for col in range(H // LANES):
            acc = row_v[0, pl.ds(col * LANES, LANES)]
            for l in range(1, LEN):
                acc = acc + row_v[l, pl.ds(col * LANES, LANES)]
            acc_v[0, pl.ds(col * LANES, LANES)] = acc * (1.0 / LEN)
        pltpu.async_copy(acc_v, out_hbm.at[pl.ds(base + r, 1)], sem).wait()
        return carry

    lax.fori_loop(0, ROWS_PER_W, body, 0)


def _sim_topk_body(xs_ref, pm_ref, sim_ref, idx_ref, rs_ref, rows_ref):
    xm = xs_ref[...] * (1.0 / T)
    xss = jnp.sum(xm * xm, axis=1, keepdims=True)
    xn = xm * lax.rsqrt(jnp.maximum(xss, 1e-12))
    pm = pm_ref[...]
    pss = jnp.sum(pm * pm, axis=1, keepdims=True)
    pn = pm * lax.rsqrt(jnp.maximum(pss, 1e-12))
    sim = lax.dot_general(
        xn, pn, (((1,), (1,)), ((), ())), preferred_element_type=jnp.float32
    )
    sim_ref[...] = sim

    iota = lax.broadcasted_iota(jnp.int32, (B, POOL), 1)
    cur = sim
    total = jnp.float32(0.0)
    cols = []
    for _ in range(K):
        m = jnp.max(cur, axis=1, keepdims=True)
        cand = jnp.where(cur == m, iota, POOL)
        i = jnp.min(cand, axis=1, keepdims=True)
        cols.append(i)
        total += jnp.sum(m)
        cur = jnp.where(iota == i, -jnp.inf, cur)
    idx_ref[...] = jnp.concatenate(cols, axis=1)
    rs_ref[0, 0] = total * (1.0 / B)

    jvec = lax.broadcasted_iota(jnp.int32, (B, LANES), 1)
    row_blocks = [cols[k] * LEN + jvec for k in range(K)]
    rows_ref[...] = jnp.concatenate(row_blocks, axis=1)


def _gather_sc_body(rows_hbm, prompt_hbm, out_hbm, idx_v, rows_v, sem):
    c = lax.axis_index("c")
    s = lax.axis_index("s")
    wid = s * NC + c
    pltpu.async_copy(rows_hbm.at[pl.ds(wid * LEN, LEN)], idx_v, sem).wait()
    pltpu.async_copy(prompt_hbm.at[idx_v], rows_v, sem).wait()
    pltpu.async_copy(rows_v, out_hbm.at[pl.ds(wid * LEN, LEN)], sem).wait()


def _stitch_body(small_ref, big_ref, out_ref):
    del big_ref
    out_ref[...] = small_ref[...]


def kernel(x_embed, prompt):
    pm = pl.kernel(
        _pmean_sc_body,
        out_type=jax.ShapeDtypeStruct((POOL, H), jnp.float32),
        mesh=plsc.VectorSubcoreMesh(core_axis_name="c", subcore_axis_name="s"),
        scratch_types=[
            pltpu.VMEM((LEN, H), jnp.float32),
            pltpu.VMEM((1, H), jnp.float32),
            pltpu.SemaphoreType.DMA,
        ],
    )(prompt)

    big0, x_sum = pl.pallas_call(
        _copy_mean_body,
        grid=(NT,),
        in_specs=[pl.BlockSpec((B, TBLK, H), lambda t: (0, t, 0))],
        out_specs=[
            pl.BlockSpec((B, TBLK, H), lambda t: (0, t + K * LEN // TBLK, 0)),
            pl.BlockSpec((B, H), lambda t: (0, 0)),
        ],
        out_shape=[
            jax.ShapeDtypeStruct((B, OUT_T, H), jnp.float32),
            jax.ShapeDtypeStruct((B, H), jnp.float32),
        ],
    )(x_embed)

    sim, idx, rs, rows = pl.pallas_call(
        _sim_topk_body,
        out_specs=[
            pl.BlockSpec(memory_space=pltpu.VMEM),
            pl.BlockSpec(memory_space=pltpu.VMEM),
            pl.BlockSpec(memory_space=pltpu.SMEM),
            pl.BlockSpec(memory_space=pltpu.VMEM),
        ],
        out_shape=[
            jax.ShapeDtypeStruct((B, POOL), jnp.float32),
            jax.ShapeDtypeStruct((B, K), jnp.int32),
            jax.ShapeDtypeStruct((1, 1), jnp.float32),
            jax.ShapeDtypeStruct((B, K * LANES), jnp.int32),
        ],
    )(x_sum, pm)

    small = pl.kernel(
        _gather_sc_body,
        out_type=jax.ShapeDtypeStruct((B * K * LEN, H), jnp.float32),
        mesh=plsc.VectorSubcoreMesh(core_axis_name="c", subcore_axis_name="s"),
        scratch_types=[
            pltpu.VMEM((LANES,), jnp.int32),
            pltpu.VMEM((LEN, H), jnp.float32),
            pltpu.SemaphoreType.DMA,
        ],
    )(rows.reshape(B * K * LEN), prompt.reshape(POOL * LEN, H))

    big = pl.pallas_call(
        _stitch_body,
        grid=(B,),
        in_specs=[
            pl.BlockSpec((1, K * LEN, H), lambda b: (b, 0, 0)),
            pl.BlockSpec((1, LEN, H), lambda b: (b, 0, 0)),
        ],
        out_specs=pl.BlockSpec((1, K * LEN, H), lambda b: (b, 0, 0)),
        out_shape=jax.ShapeDtypeStruct((B, OUT_T, H), jnp.float32),
        input_output_aliases={1: 0},
    )(small.reshape(B, K * LEN, H), big0)

    return big, rs[0, 0], sim, idx


# speedup vs baseline: 1.2287x; 1.2287x over previous
import functools

import jax
import jax.numpy as jnp
from jax import lax
from jax.experimental import pallas as pl
from jax.experimental.pallas import tpu as pltpu
from jax.experimental.pallas import tpu_sc as plsc

POOL = 1024
LEN = 16
K = 8
B, T, H = 4, 8192, 1024

TBLK = 128
NT = T // TBLK
OUT_T = K * LEN + T

NC = 2
NS = 16
NW = NC * NS
ROWS_PER_W = POOL // NW
LANES = 16


def _copy_mean_body(x_ref, out_ref, sum_ref):
    t = pl.program_id(0)
    xb = x_ref[...]
    out_ref[...] = xb

    @pl.when(t == 0)
    def _():
        sum_ref[...] = jnp.zeros_like(sum_ref)

    sum_ref[...] += jnp.sum(xb, axis=1)


GROUPS = 8
GROW = ROWS_PER_W // GROUPS


def _pmean_sc_body(prompt_hbm, out_hbm, buf_v, outb_v, sem):
    c = lax.axis_index("c")
    s = lax.axis_index("s")
    wid = s * NC + c
    base = wid * ROWS_PER_W

    def group(g, carry):
        pltpu.async_copy(
            prompt_hbm.at[pl.ds((base + g * GROW) * LEN, GROW * LEN)], buf_v, sem
        ).wait()
        for b in range(GROW):

            @plsc.parallel_loop(0, H // LANES, unroll=4)
            def _chunk(col):
                off = col * LANES
                acc = buf_v[b * LEN, pl.ds(off, LANES)]
                for l in range(1, LEN):
                    acc = acc + buf_v[b * LEN + l, pl.ds(off, LANES)]
                outb_v[g * GROW + b, pl.ds(off, LANES)] = acc * (1.0 / LEN)

        return carry

    lax.fori_loop(0, GROUPS, group, 0)
    pltpu.async_copy(outb_v, out_hbm.at[pl.ds(base, ROWS_PER_W)], sem).wait()


def _sim_topk_body(xs_ref, pm_ref, sim_ref, idx_ref, rs_ref, rows_ref):
    xm = xs_ref[...] * (1.0 / T)
    xss = jnp.sum(xm * xm, axis=1, keepdims=True)
    xn = xm * lax.rsqrt(jnp.maximum(xss, 1e-12))
    pm = pm_ref[...]
    pss = jnp.sum(pm * pm, axis=1, keepdims=True)
    pn = pm * lax.rsqrt(jnp.maximum(pss, 1e-12))
    sim = lax.dot_general(
        xn, pn, (((1,), (1,)), ((), ())), preferred_element_type=jnp.float32
    )
    sim_ref[...] = sim

    iota = lax.broadcasted_iota(jnp.int32, (B, POOL), 1)
    cur = sim
    total = jnp.float32(0.0)
    cols = []
    for _ in range(K):
        m = jnp.max(cur, axis=1, keepdims=True)
        cand = jnp.where(cur == m, iota, POOL)
        i = jnp.min(cand, axis=1, keepdims=True)
        cols.append(i)
        total += jnp.sum(m)
        cur = jnp.where(iota == i, -jnp.inf, cur)
    idx_ref[...] = jnp.concatenate(cols, axis=1)
    rs_ref[0, 0] = total * (1.0 / B)

    jvec = lax.broadcasted_iota(jnp.int32, (B, LANES), 1)
    row_blocks = [cols[k] * LEN + jvec for k in range(K)]
    rows_ref[...] = jnp.concatenate(row_blocks, axis=1)


def _gather_sc_body(rows_hbm, prompt_hbm, out_hbm, idx_v, rows_v, sem):
    c = lax.axis_index("c")
    s = lax.axis_index("s")
    wid = s * NC + c
    pltpu.async_copy(rows_hbm.at[pl.ds(wid * LEN, LEN)], idx_v, sem).wait()
    pltpu.async_copy(prompt_hbm.at[idx_v], rows_v, sem).wait()
    pltpu.async_copy(rows_v, out_hbm.at[pl.ds(wid * LEN, LEN)], sem).wait()


def _stitch_body(small_ref, big_ref, out_ref):
    del big_ref
    out_ref[...] = small_ref[...]


def kernel(x_embed, prompt):
    pm = pl.kernel(
        _pmean_sc_body,
        out_type=jax.ShapeDtypeStruct((POOL, H), jnp.float32),
        mesh=plsc.VectorSubcoreMesh(core_axis_name="c", subcore_axis_name="s"),
        scratch_types=[
            pltpu.VMEM((GROW * LEN, H), jnp.float32),
            pltpu.VMEM((ROWS_PER_W, H), jnp.float32),
            pltpu.SemaphoreType.DMA,
        ],
    )(prompt.reshape(POOL * LEN, H))

    big0, x_sum = pl.pallas_call(
        _copy_mean_body,
        grid=(NT,),
        in_specs=[pl.BlockSpec((B, TBLK, H), lambda t: (0, t, 0))],
        out_specs=[
            pl.BlockSpec((B, TBLK, H), lambda t: (0, t + K * LEN // TBLK, 0)),
            pl.BlockSpec((B, H), lambda t: (0, 0)),
        ],
        out_shape=[
            jax.ShapeDtypeStruct((B, OUT_T, H), jnp.float32),
            jax.ShapeDtypeStruct((B, H), jnp.float32),
        ],
    )(x_embed)

    sim, idx, rs, rows = pl.pallas_call(
        _sim_topk_body,
        out_specs=[
            pl.BlockSpec(memory_space=pltpu.VMEM),
            pl.BlockSpec(memory_space=pltpu.VMEM),
            pl.BlockSpec(memory_space=pltpu.SMEM),
            pl.BlockSpec(memory_space=pltpu.VMEM),
        ],
        out_shape=[
            jax.ShapeDtypeStruct((B, POOL), jnp.float32),
            jax.ShapeDtypeStruct((B, K), jnp.int32),
            jax.ShapeDtypeStruct((1, 1), jnp.float32),
            jax.ShapeDtypeStruct((B, K * LANES), jnp.int32),
        ],
    )(x_sum, pm)

    small = pl.kernel(
        _gather_sc_body,
        out_type=jax.ShapeDtypeStruct((B * K * LEN, H), jnp.float32),
        mesh=plsc.VectorSubcoreMesh(core_axis_name="c", subcore_axis_name="s"),
        scratch_types=[
            pltpu.VMEM((LANES,), jnp.int32),
            pltpu.VMEM((LEN, H), jnp.float32),
            pltpu.SemaphoreType.DMA,
        ],
    )(rows.reshape(B * K * LEN), prompt.reshape(POOL * LEN, H))

    big = pl.pallas_call(
        _stitch_body,
        grid=(B,),
        in_specs=[
            pl.BlockSpec((1, K * LEN, H), lambda b: (b, 0, 0)),
            pl.BlockSpec((1, LEN, H), lambda b: (b, 0, 0)),
        ],
        out_specs=pl.BlockSpec((1, K * LEN, H), lambda b: (b, 0, 0)),
        out_shape=jax.ShapeDtypeStruct((B, OUT_T, H), jnp.float32),
        input_output_aliases={1: 0},
    )(small.reshape(B, K * LEN, H), big0)

    return big, rs[0, 0], sim, idx


# speedup vs baseline: 1.2859x; 1.0466x over previous
import functools

import jax
import jax.numpy as jnp
from jax import lax
from jax.experimental import pallas as pl
from jax.experimental.pallas import tpu as pltpu
from jax.experimental.pallas import tpu_sc as plsc

POOL = 1024
LEN = 16
K = 8
B, T, H = 4, 8192, 1024

TBLK = 512
NT = T // TBLK
OUT_T = K * LEN + T

NC = 2
NS = 16
NW = NC * NS
ROWS_PER_W = POOL // NW
LANES = 16


def _copy_mean_body(x_ref, sum_ref, big_ref, sem):
    t = pl.program_id(0)

    cp = pltpu.make_async_copy(
        x_ref, big_ref.at[:, pl.ds(K * LEN + t * TBLK, TBLK), :], sem
    )
    cp.start()

    @pl.when(t == 0)
    def _():
        sum_ref[...] = jnp.zeros_like(sum_ref)

    sum_ref[...] += jnp.sum(x_ref[...], axis=1)

    cp.wait()


GROUPS = 8
GROW = ROWS_PER_W // GROUPS


def _pmean_sc_body(prompt_hbm, out_hbm, buf_v, outb_v, sem):
    c = lax.axis_index("c")
    s = lax.axis_index("s")
    wid = s * NC + c
    base = wid * ROWS_PER_W

    def group(g, carry):
        pltpu.async_copy(
            prompt_hbm.at[pl.ds((base + g * GROW) * LEN, GROW * LEN)], buf_v, sem
        ).wait()
        for b in range(GROW):

            @plsc.parallel_loop(0, H // LANES, unroll=4)
            def _chunk(col):
                off = col * LANES
                acc = buf_v[b * LEN, pl.ds(off, LANES)]
                for l in range(1, LEN):
                    acc = acc + buf_v[b * LEN + l, pl.ds(off, LANES)]
                outb_v[g * GROW + b, pl.ds(off, LANES)] = acc * (1.0 / LEN)

        return carry

    lax.fori_loop(0, GROUPS, group, 0)
    pltpu.async_copy(outb_v, out_hbm.at[pl.ds(base, ROWS_PER_W)], sem).wait()


def _sim_topk_body(xs_ref, pm_ref, sim_ref, idx_ref, rs_ref, rows_ref):
    xm = xs_ref[...] * (1.0 / T)
    xss = jnp.sum(xm * xm, axis=1, keepdims=True)
    xn = xm * lax.rsqrt(jnp.maximum(xss, 1e-12))
    pm = pm_ref[...]
    pss = jnp.sum(pm * pm, axis=1, keepdims=True)
    pn = pm * lax.rsqrt(jnp.maximum(pss, 1e-12))
    sim = lax.dot_general(
        xn, pn, (((1,), (1,)), ((), ())), preferred_element_type=jnp.float32
    )
    sim_ref[...] = sim

    iota = lax.broadcasted_iota(jnp.int32, (B, POOL), 1)
    cur = sim
    total = jnp.float32(0.0)
    cols = []
    for _ in range(K):
        m = jnp.max(cur, axis=1, keepdims=True)
        cand = jnp.where(cur == m, iota, POOL)
        i = jnp.min(cand, axis=1, keepdims=True)
        cols.append(i)
        total += jnp.sum(m)
        cur = jnp.where(iota == i, -jnp.inf, cur)
    idx_ref[...] = jnp.concatenate(cols, axis=1)
    rs_ref[0, 0] = total * (1.0 / B)

    jvec = lax.broadcasted_iota(jnp.int32, (B, LANES), 1)
    row_blocks = [cols[k] * LEN + jvec for k in range(K)]
    rows_ref[...] = jnp.concatenate(row_blocks, axis=1)


def _gather_sc_body(rows_hbm, prompt_hbm, out_hbm, idx_v, rows_v, sem):
    c = lax.axis_index("c")
    s = lax.axis_index("s")
    wid = s * NC + c
    pltpu.async_copy(rows_hbm.at[pl.ds(wid * LEN, LEN)], idx_v, sem).wait()
    pltpu.async_copy(prompt_hbm.at[idx_v], rows_v, sem).wait()
    pltpu.async_copy(rows_v, out_hbm.at[pl.ds(wid * LEN, LEN)], sem).wait()


def _stitch_body(small_ref, big_ref, out_ref):
    del big_ref
    out_ref[...] = small_ref[...]


def kernel(x_embed, prompt):
    pm = pl.kernel(
        _pmean_sc_body,
        out_type=jax.ShapeDtypeStruct((POOL, H), jnp.float32),
        mesh=plsc.VectorSubcoreMesh(core_axis_name="c", subcore_axis_name="s"),
        scratch_types=[
            pltpu.VMEM((GROW * LEN, H), jnp.float32),
            pltpu.VMEM((ROWS_PER_W, H), jnp.float32),
            pltpu.SemaphoreType.DMA,
        ],
    )(prompt.reshape(POOL * LEN, H))

    x_sum, big0 = pl.pallas_call(
        _copy_mean_body,
        grid=(NT,),
        in_specs=[pl.BlockSpec((B, TBLK, H), lambda t: (0, t, 0))],
        out_specs=[
            pl.BlockSpec((B, H), lambda t: (0, 0)),
            pl.BlockSpec(memory_space=pl.ANY),
        ],
        out_shape=[
            jax.ShapeDtypeStruct((B, H), jnp.float32),
            jax.ShapeDtypeStruct((B, OUT_T, H), jnp.float32),
        ],
        scratch_shapes=[pltpu.SemaphoreType.DMA],
    )(x_embed)

    sim, idx, rs, rows = pl.pallas_call(
        _sim_topk_body,
        out_specs=[
            pl.BlockSpec(memory_space=pltpu.VMEM),
            pl.BlockSpec(memory_space=pltpu.VMEM),
            pl.BlockSpec(memory_space=pltpu.SMEM),
            pl.BlockSpec(memory_space=pltpu.VMEM),
        ],
        out_shape=[
            jax.ShapeDtypeStruct((B, POOL), jnp.float32),
            jax.ShapeDtypeStruct((B, K), jnp.int32),
            jax.ShapeDtypeStruct((1, 1), jnp.float32),
            jax.ShapeDtypeStruct((B, K * LANES), jnp.int32),
        ],
    )(x_sum, pm)

    small = pl.kernel(
        _gather_sc_body,
        out_type=jax.ShapeDtypeStruct((B * K * LEN, H), jnp.float32),
        mesh=plsc.VectorSubcoreMesh(core_axis_name="c", subcore_axis_name="s"),
        scratch_types=[
            pltpu.VMEM((LANES,), jnp.int32),
            pltpu.VMEM((LEN, H), jnp.float32),
            pltpu.SemaphoreType.DMA,
        ],
    )(rows.reshape(B * K * LEN), prompt.reshape(POOL * LEN, H))

    big = pl.pallas_call(
        _stitch_body,
        grid=(B,),
        in_specs=[
            pl.BlockSpec((1, K * LEN, H), lambda b: (b, 0, 0)),
            pl.BlockSpec((1, LEN, H), lambda b: (b, 0, 0)),
        ],
        out_specs=pl.BlockSpec((1, K * LEN, H), lambda b: (b, 0, 0)),
        out_shape=jax.ShapeDtypeStruct((B, OUT_T, H), jnp.float32),
        input_output_aliases={1: 0},
    )(small.reshape(B, K * LEN, H), big0)

    return big, rs[0, 0], sim, idx
